# baseline (speedup 1.0000x reference)
import functools

import jax
import jax.numpy as jnp
from jax import lax
from jax.experimental import pallas as pl
from jax.experimental.pallas import tpu as pltpu
from jax.experimental.pallas import tpu_sc as plsc

D = 64
DP = 128
NW = 32
CHUNK = 128
NBUF = 3
UNROLL = 3


def _build_gather(seq: int, batch: int):
    nb = batch // CHUNK
    assert nb == NW
    nloop = seq // UNROLL
    mesh = plsc.VectorSubcoreMesh(core_axis_name="c", subcore_axis_name="s")

    @functools.partial(
        pl.kernel,
        mesh=mesh,
        compiler_params=pltpu.CompilerParams(needs_layout_passes=False),
        out_type=jax.ShapeDtypeStruct((seq * batch, DP), jnp.float32),
        scratch_types=[
            pltpu.VMEM((seq, CHUNK), jnp.int32),
            [pltpu.VMEM((CHUNK, DP), jnp.float32) for _ in range(NBUF)],
            [pltpu.SemaphoreType.DMA for _ in range(NBUF)],
            [pltpu.SemaphoreType.DMA for _ in range(NBUF)],
        ],
    )
    def emb(table_hbm, idx_hbm, out_hbm, idx_v, bufs, gsems, psems):
        wid = lax.axis_index("s") * 2 + lax.axis_index("c")
        base = wid * seq * CHUNK
        pltpu.sync_copy(idx_hbm.at[wid], idx_v)

        def gather(s, b):
            pltpu.async_copy(table_hbm.at[idx_v.at[s]], bufs[b], gsems[b])

        def gather_wait(s, b):
            pltpu.make_async_copy(
                table_hbm.at[idx_v.at[s]], bufs[b], gsems[b]).wait()

        def put(s, b):
            pltpu.async_copy(bufs[b],
                             out_hbm.at[pl.ds(base + s * CHUNK, CHUNK)],
                             psems[b])

        def put_wait(s, b):
            pltpu.make_async_copy(
                bufs[b],
                out_hbm.at[pl.ds(base + s * CHUNK, CHUNK)], psems[b]).wait()

        def chunk_step(s, k, first, refill):
            b = k % NBUF
            bp = (k + NBUF - 1) % NBUF
            gather_wait(s, b)
            put(s, b)
            if not first:
                put_wait(s - 1, bp)
            if refill == "static":
                gather(s + 2, (k + 2) % NBUF)
            elif refill == "guarded":
                @pl.when(s + 2 < seq)
                def _():
                    gather(s + 2, (k + 2) % NBUF)

        for b in range(2):
            gather(b, b)

        def body(p, carry):
            s0 = p * UNROLL
            for k in range(UNROLL):
                chunk_step(s0 + k, k, False, "guarded")
            return carry

        for k in range(UNROLL):
            chunk_step(k, k, k == 0, "static")
        lax.fori_loop(1, nloop, body, 0)
        tail0 = nloop * UNROLL
        for t in range(seq - tail0):
            chunk_step(tail0 + t, t, False, "none")
        put_wait(seq - 1, (seq - 1) % NBUF)

    return emb


def kernel(input_ids, attention_mask, table):
    b, s = input_ids.shape
    table_pad = jnp.pad(table, ((0, 0), (0, DP - D)))
    ids_w = input_ids.reshape(NW, (b * s) // (NW * CHUNK), CHUNK).astype(jnp.int32)
    out2 = _build_gather(s, b)(table_pad, ids_w)
    return out2[:, :D].reshape(b, s, D), attention_mask
